# baseline (speedup 1.0000x reference)
import functools

import jax
import jax.numpy as jnp
from jax import lax
from jax.experimental import pallas as pl
from jax.experimental.pallas import tpu as pltpu
from jax.experimental.pallas import tpu_sc as plsc

_N = 10000
_E = 320000
_C = 128
_CH2 = _C // 2
_NS = 16
_NC = 2
_K = 125
_CH = 80
_SEC = 10000
_NPAD = 10240
_TROWS = _NPAD // _NS


def _leaky(e):
    return jnp.where(e >= 0.0, e, 0.2 * e)


def _sc_body(ei_hbm, ei2_hbm, av_hbm, h_hbm,
             out_hbm,
             asrc_v, adst_v, denom_v, srcf_v, dstf_v, src2_v, dst2_v,
             rb0, rb1, exch0, exch1, alpha_v, zero_v,
             denom_s, aggr_s,
             sem_g0, sem_g1, sem_s0, sem_s1, sem_d0, sem_d1):
    c = lax.axis_index("c")
    s = lax.axis_index("s")
    rb = (rb0, rb1)
    exch = (exch0, exch1)
    sem_g = (sem_g0, sem_g1)
    sem_s = (sem_s0, sem_s1)
    sem_d = (sem_d0, sem_d1)

    def _zrow(i, _):
        rb0[i // 4, pl.ds((i % 4) * 16, 16)] = jnp.zeros((16,), jnp.float32)
        return 0
    lax.fori_loop(0, 128 * 4, _zrow, 0)

    def _zv(i, _):
        zero_v[pl.ds(i * 16, 16)] = jnp.zeros((16,), jnp.float32)
        return 0
    lax.fori_loop(0, 64, _zv, 0)

    for m in range(5):
        pltpu.sync_copy(rb0, aggr_s.at[pl.ds(s * _TROWS + m * 128, 128)])

    @pl.when(s == 0)
    def _():
        for m in range(10):
            pltpu.sync_copy(zero_v.at[pl.ds(0, 1000)],
                            denom_s.at[pl.ds(m * 1000, 1000)])

    pltpu.sync_copy(av_hbm.at[0], asrc_v)
    pltpu.sync_copy(av_hbm.at[1], adst_v)

    plsc.subcore_barrier()

    def _section_load(t):
        base = s * (2 * _SEC) + t * _SEC
        pltpu.sync_copy(ei_hbm.at[0].at[pl.ds(base, _SEC)],
                        srcf_v.at[pl.ds(0, _SEC)])
        pltpu.sync_copy(ei_hbm.at[1].at[pl.ds(base, _SEC)],
                        dstf_v.at[pl.ds(0, _SEC)])
        srcf_v[pl.ds(_SEC, 16)] = jnp.zeros((16,), jnp.int32)
        dstf_v[pl.ds(_SEC, 16)] = jnp.zeros((16,), jnp.int32)
        row0 = s * 160 + t * _CH
        pltpu.sync_copy(ei2_hbm.at[0].at[pl.ds(row0, _CH)], src2_v)
        pltpu.sync_copy(ei2_hbm.at[1].at[pl.ds(row0, _CH)], dst2_v)

    def _ex_vecs(j, dst_buf):
        @plsc.parallel_loop(0, 8, unroll=4)
        def _vec(tt):
            off = j * _K + tt * 16
            s16 = srcf_v[pl.ds(off, 16)]
            d16 = dstf_v[pl.ds(off, 16)]
            e = (plsc.load_gather(asrc_v, [s16])
                 + plsc.load_gather(adst_v, [d16]))
            dst_buf[pl.ds(tt * 16, 16)] = jnp.exp(_leaky(e))

    for t in range(2):
        _section_load(t)

        def _dchunk(jo, _):
            for b in range(2):
                j = 2 * jo + b

                @pl.when(j >= 2)
                def _():
                    pltpu.make_async_copy(
                        exch[b].at[pl.ds(0, _K)],
                        denom_s.at[dst2_v.at[j - 2]], sem_d[b]).wait()
                _ex_vecs(j, exch[b])
                pltpu.async_copy(exch[b].at[pl.ds(0, _K)],
                                 denom_s.at[dst2_v.at[j]], sem_d[b],
                                 add=True)
            return 0
        lax.fori_loop(0, _CH // 2, _dchunk, 0)
        for b in range(2):
            pltpu.make_async_copy(exch[b].at[pl.ds(0, _K)],
                                  denom_s.at[dst2_v.at[_CH - 2 + b]],
                                  sem_d[b]).wait()

    plsc.subcore_barrier()

    pltpu.sync_copy(denom_s, denom_v)
    h_c = h_hbm.at[c]

    def _alpha_vecs(j):
        @plsc.parallel_loop(0, 8, unroll=4)
        def _vec(tt):
            off = j * _K + tt * 16
            s16 = srcf_v[pl.ds(off, 16)]
            d16 = dstf_v[pl.ds(off, 16)]
            e = (plsc.load_gather(asrc_v, [s16])
                 + plsc.load_gather(adst_v, [d16]))
            ex = jnp.exp(_leaky(e))
            den = plsc.load_gather(denom_v, [d16])
            alpha_v[pl.ds(tt * 16, 16)] = ex / (den + 1e-16)

    for t in range(2):
        _section_load(t)
        pltpu.async_copy(h_c.at[src2_v.at[0]], rb0.at[pl.ds(0, _K)], sem_g0)

        def _rchunk(jo, _):
            for b in range(2):
                j = 2 * jo + b
                _alpha_vecs(j)

                @pl.when(j >= 1)
                def _():
                    pltpu.make_async_copy(
                        rb[1 - b].at[pl.ds(0, _K)],
                        aggr_s.at[dst2_v.at[j - 1]], sem_s[1 - b]).wait()

                @pl.when(j <= _CH - 2)
                def _():
                    pltpu.async_copy(h_c.at[src2_v.at[j + 1]],
                                     rb[1 - b].at[pl.ds(0, _K)],
                                     sem_g[1 - b])

                pltpu.make_async_copy(h_c.at[src2_v.at[j]],
                                      rb[b].at[pl.ds(0, _K)],
                                      sem_g[b]).wait()

                rbb = rb[b]

                @plsc.parallel_loop(0, 8, unroll=4)
                def _scale(g):
                    a16 = alpha_v[pl.ds(g * 16, 16)]
                    for ll in range(16):
                        r = g * 16 + ll
                        af = jnp.full((16,), a16[ll], jnp.float32)
                        for k2 in range(4):
                            rbb[r, pl.ds(k2 * 16, 16)] = (
                                rbb[r, pl.ds(k2 * 16, 16)] * af)

                pltpu.async_copy(rb[b].at[pl.ds(0, _K)],
                                 aggr_s.at[dst2_v.at[j]], sem_s[b], add=True)
            return 0
        lax.fori_loop(0, _CH // 2, _rchunk, 0)
        pltpu.make_async_copy(rb[1].at[pl.ds(0, _K)],
                              aggr_s.at[dst2_v.at[_CH - 1]],
                              sem_s[1]).wait()

    plsc.subcore_barrier()

    pltpu.sync_copy(aggr_s.at[pl.ds(s * _TROWS, _TROWS)],
                    out_hbm.at[c].at[pl.ds(s * _TROWS, _TROWS)])


_sc_edges = functools.partial(
    pl.kernel,
    out_type=jax.ShapeDtypeStruct((_NC, _NPAD, _CH2), jnp.float32),
    mesh=plsc.VectorSubcoreMesh(core_axis_name="c", subcore_axis_name="s"),
    compiler_params=pltpu.CompilerParams(needs_layout_passes=False,
                                         use_tc_tiling_on_sc=False),
    scratch_types=[
        pltpu.VMEM((_N,), jnp.float32),
        pltpu.VMEM((_N,), jnp.float32),
        pltpu.VMEM((_N,), jnp.float32),
        pltpu.VMEM((_SEC + 16,), jnp.int32),
        pltpu.VMEM((_SEC + 16,), jnp.int32),
        pltpu.VMEM((_CH, _K), jnp.int32),
        pltpu.VMEM((_CH, _K), jnp.int32),
        pltpu.VMEM((128, _CH2), jnp.float32),
        pltpu.VMEM((128, _CH2), jnp.float32),
        pltpu.VMEM((128,), jnp.float32),
        pltpu.VMEM((128,), jnp.float32),
        pltpu.VMEM((144,), jnp.float32),
        pltpu.VMEM((1024,), jnp.float32),
        pltpu.VMEM_SHARED((_N,), jnp.float32),
        pltpu.VMEM_SHARED((_NPAD, _CH2), jnp.float32),
        pltpu.SemaphoreType.DMA,
        pltpu.SemaphoreType.DMA,
        pltpu.SemaphoreType.DMA,
        pltpu.SemaphoreType.DMA,
        pltpu.SemaphoreType.DMA,
        pltpu.SemaphoreType.DMA,
    ],
)(_sc_body)


_BLK = 1000


def _emit_h_av(h, as_ref, ad_ref, h_ref, av_ref):
    h_ref[0] = h[:, :_CH2]
    h_ref[1] = h[:, _CH2:]
    av_ref[0, :, 0] = jnp.sum(h * as_ref[...], axis=1)
    av_ref[1, :, 0] = jnp.sum(h * ad_ref[...], axis=1)


def _tc_first_body(x_ref, w0_ref, b0_ref, w1_ref, as_ref, ad_ref,
                   out_ref, h_ref, av_ref):
    t = jnp.dot(x_ref[...], w0_ref[...],
                preferred_element_type=jnp.float32) + b0_ref[...]
    o = jax.nn.softplus(t)
    out_ref[...] = o
    h = jnp.dot(o, w1_ref[...], preferred_element_type=jnp.float32)
    _emit_h_av(h, as_ref, ad_ref, h_ref, av_ref)


def _tc_mid_body(o_ref, a_ref, b_ref, w_ref, as_ref, ad_ref,
                 out_ref, h_ref, av_ref):
    o = (o_ref[...] + jnp.concatenate([a_ref[0], a_ref[1]], axis=-1)
         + b_ref[...])
    out_ref[...] = o
    h = jnp.dot(o, w_ref[...], preferred_element_type=jnp.float32)
    _emit_h_av(h, as_ref, ad_ref, h_ref, av_ref)


def _tc_final_body(o_ref, a_ref, b_ref, out_ref):
    out_ref[...] = (o_ref[...]
                    + jnp.concatenate([a_ref[0], a_ref[1]], axis=-1)
                    + b_ref[...])


def _row_spec():
    return pl.BlockSpec((_BLK, _C), lambda i: (i, 0))


def _aggr_spec():
    return pl.BlockSpec((2, _BLK, _CH2), lambda i: (0, i, 0))


def _h_spec():
    return pl.BlockSpec((2, _BLK, _CH2), lambda i: (0, i, 0))


def _av_spec():
    return pl.BlockSpec((2, _BLK, 1), lambda i: (0, i, 0))


def _w_spec():
    return pl.BlockSpec((_C, _C), lambda i: (0, 0))


def _b_spec():
    return pl.BlockSpec((1, _C), lambda i: (0, 0))


_out_h_av_shapes = [jax.ShapeDtypeStruct((_N, _C), jnp.float32),
                    jax.ShapeDtypeStruct((2, _N, _CH2), jnp.float32),
                    jax.ShapeDtypeStruct((2, _N, 1), jnp.float32)]

_tc_first = pl.pallas_call(
    _tc_first_body,
    grid=(_N // _BLK,),
    in_specs=[_row_spec(), _w_spec(), _b_spec(), _w_spec(),
              _b_spec(), _b_spec()],
    out_specs=[_row_spec(), _h_spec(), _av_spec()],
    out_shape=_out_h_av_shapes,
)

_tc_mid = pl.pallas_call(
    _tc_mid_body,
    grid=(_N // _BLK,),
    in_specs=[_row_spec(), _aggr_spec(), _b_spec(), _w_spec(),
              _b_spec(), _b_spec()],
    out_specs=[_row_spec(), _h_spec(), _av_spec()],
    out_shape=_out_h_av_shapes,
)

_tc_final = pl.pallas_call(
    _tc_final_body,
    grid=(_N // _BLK,),
    in_specs=[_row_spec(), _aggr_spec(), _b_spec()],
    out_specs=_row_spec(),
    out_shape=jax.ShapeDtypeStruct((_N, _C), jnp.float32),
)


def kernel(x, edge_index, edge_weight, edge_attr, lin0_W, lin0_b,
           conv_W, att_src, att_dst, conv_bias):
    del edge_weight, edge_attr
    ei = edge_index.astype(jnp.int32)
    ei2 = ei.reshape(2, _E // _K, _K)

    out0, h1, av1 = _tc_first(x, lin0_W, lin0_b.reshape(1, _C), conv_W[0],
                              att_src[0].reshape(1, _C),
                              att_dst[0].reshape(1, _C))
    aggr1 = _sc_edges(ei, ei2, av1.reshape(2, _N), h1)
    out1, h2, av2 = _tc_mid(out0, aggr1, conv_bias[0].reshape(1, _C),
                            conv_W[1],
                            att_src[1].reshape(1, _C),
                            att_dst[1].reshape(1, _C))
    aggr2 = _sc_edges(ei, ei2, av2.reshape(2, _N), h2)
    return _tc_final(out1, aggr2, conv_bias[1].reshape(1, _C))


# speedup vs baseline: 1.0188x; 1.0188x over previous
import functools

import jax
import jax.numpy as jnp
from jax import lax
from jax.experimental import pallas as pl
from jax.experimental.pallas import tpu as pltpu
from jax.experimental.pallas import tpu_sc as plsc

_N = 10000
_E = 320000
_C = 128
_CH2 = _C // 2
_NS = 16
_NC = 2
_K = 125
_CH = 80
_SEC = 10000
_NPAD = 10240
_TROWS = _NPAD // _NS


def _leaky(e):
    return jnp.where(e >= 0.0, e, 0.2 * e)


def _sc_body(ei_hbm, ei2_hbm, av_hbm, h_hbm,
             out_hbm,
             asrc_v, adst_v, denom_v, srcf_v, dstf_v, src2_v, dst2_v,
             rb0, rb1, exch0, exch1, exch2, exch3, alpha_v, zero_v,
             denom_s, aggr_s,
             sem_g0, sem_g1, sem_s0, sem_s1,
             sem_d0, sem_d1, sem_d2, sem_d3):
    c = lax.axis_index("c")
    s = lax.axis_index("s")
    rb = (rb0, rb1)
    exch = (exch0, exch1, exch2, exch3)
    sem_g = (sem_g0, sem_g1)
    sem_s = (sem_s0, sem_s1)
    sem_d = (sem_d0, sem_d1, sem_d2, sem_d3)

    def _zrow(i, _):
        rb0[i // 4, pl.ds((i % 4) * 16, 16)] = jnp.zeros((16,), jnp.float32)
        return 0
    lax.fori_loop(0, 128 * 4, _zrow, 0)

    def _zv(i, _):
        zero_v[pl.ds(i * 16, 16)] = jnp.zeros((16,), jnp.float32)
        return 0
    lax.fori_loop(0, 64, _zv, 0)

    for m in range(5):
        pltpu.sync_copy(rb0, aggr_s.at[pl.ds(s * _TROWS + m * 128, 128)])

    @pl.when(s == 0)
    def _():
        for m in range(10):
            pltpu.sync_copy(zero_v.at[pl.ds(0, 1000)],
                            denom_s.at[pl.ds(m * 1000, 1000)])

    pltpu.sync_copy(av_hbm.at[0], asrc_v)
    pltpu.sync_copy(av_hbm.at[1], adst_v)

    plsc.subcore_barrier()

    def _section_load(t):
        base = s * (2 * _SEC) + t * _SEC
        pltpu.sync_copy(ei_hbm.at[0].at[pl.ds(base, _SEC)],
                        srcf_v.at[pl.ds(0, _SEC)])
        pltpu.sync_copy(ei_hbm.at[1].at[pl.ds(base, _SEC)],
                        dstf_v.at[pl.ds(0, _SEC)])
        srcf_v[pl.ds(_SEC, 16)] = jnp.zeros((16,), jnp.int32)
        dstf_v[pl.ds(_SEC, 16)] = jnp.zeros((16,), jnp.int32)
        row0 = s * 160 + t * _CH
        pltpu.sync_copy(ei2_hbm.at[0].at[pl.ds(row0, _CH)], src2_v)
        pltpu.sync_copy(ei2_hbm.at[1].at[pl.ds(row0, _CH)], dst2_v)

    def _ex_vecs(j, dst_buf):
        @plsc.parallel_loop(0, 8, unroll=2)
        def _vec(tt):
            off = j * _K + tt * 16
            s16 = srcf_v[pl.ds(off, 16)]
            d16 = dstf_v[pl.ds(off, 16)]
            e = (plsc.load_gather(asrc_v, [s16])
                 + plsc.load_gather(adst_v, [d16]))
            dst_buf[pl.ds(tt * 16, 16)] = jnp.exp(_leaky(e))

    for t in range(2):
        _section_load(t)

        def _dchunk(jo, _):
            for b in range(4):
                j = 4 * jo + b

                @pl.when(j >= 4)
                def _():
                    pltpu.make_async_copy(
                        exch[b].at[pl.ds(0, _K)],
                        denom_s.at[dst2_v.at[j - 4]], sem_d[b]).wait()
                _ex_vecs(j, exch[b])
                pltpu.async_copy(exch[b].at[pl.ds(0, _K)],
                                 denom_s.at[dst2_v.at[j]], sem_d[b],
                                 add=True)
            return 0
        lax.fori_loop(0, _CH // 4, _dchunk, 0)
        for b in range(4):
            pltpu.make_async_copy(exch[b].at[pl.ds(0, _K)],
                                  denom_s.at[dst2_v.at[_CH - 4 + b]],
                                  sem_d[b]).wait()

    plsc.subcore_barrier()

    pltpu.sync_copy(denom_s, denom_v)
    h_c = h_hbm.at[c]

    def _alpha_vecs(j):
        @plsc.parallel_loop(0, 8, unroll=2)
        def _vec(tt):
            off = j * _K + tt * 16
            s16 = srcf_v[pl.ds(off, 16)]
            d16 = dstf_v[pl.ds(off, 16)]
            e = (plsc.load_gather(asrc_v, [s16])
                 + plsc.load_gather(adst_v, [d16]))
            ex = jnp.exp(_leaky(e))
            den = plsc.load_gather(denom_v, [d16])
            alpha_v[pl.ds(tt * 16, 16)] = ex / (den + 1e-16)

    for t in range(2):
        _section_load(t)
        pltpu.async_copy(h_c.at[src2_v.at[0]], rb0.at[pl.ds(0, _K)], sem_g0)

        def _rchunk(jo, _):
            for b in range(2):
                j = 2 * jo + b
                _alpha_vecs(j)

                @pl.when(j >= 1)
                def _():
                    pltpu.make_async_copy(
                        rb[1 - b].at[pl.ds(0, _K)],
                        aggr_s.at[dst2_v.at[j - 1]], sem_s[1 - b]).wait()

                @pl.when(j <= _CH - 2)
                def _():
                    pltpu.async_copy(h_c.at[src2_v.at[j + 1]],
                                     rb[1 - b].at[pl.ds(0, _K)],
                                     sem_g[1 - b])

                pltpu.make_async_copy(h_c.at[src2_v.at[j]],
                                      rb[b].at[pl.ds(0, _K)],
                                      sem_g[b]).wait()

                rbb = rb[b]

                @plsc.parallel_loop(0, 8, unroll=2)
                def _scale(g):
                    a16 = alpha_v[pl.ds(g * 16, 16)]
                    for ll in range(16):
                        r = g * 16 + ll
                        af = jnp.full((16,), a16[ll], jnp.float32)
                        for k2 in range(4):
                            rbb[r, pl.ds(k2 * 16, 16)] = (
                                rbb[r, pl.ds(k2 * 16, 16)] * af)

                pltpu.async_copy(rb[b].at[pl.ds(0, _K)],
                                 aggr_s.at[dst2_v.at[j]], sem_s[b], add=True)
            return 0
        lax.fori_loop(0, _CH // 2, _rchunk, 0)
        pltpu.make_async_copy(rb[1].at[pl.ds(0, _K)],
                              aggr_s.at[dst2_v.at[_CH - 1]],
                              sem_s[1]).wait()

    plsc.subcore_barrier()

    pltpu.sync_copy(aggr_s.at[pl.ds(s * _TROWS, _TROWS)],
                    out_hbm.at[c].at[pl.ds(s * _TROWS, _TROWS)])


_sc_edges = functools.partial(
    pl.kernel,
    out_type=jax.ShapeDtypeStruct((_NC, _NPAD, _CH2), jnp.float32),
    mesh=plsc.VectorSubcoreMesh(core_axis_name="c", subcore_axis_name="s"),
    compiler_params=pltpu.CompilerParams(needs_layout_passes=False,
                                         use_tc_tiling_on_sc=False),
    scratch_types=[
        pltpu.VMEM((_N,), jnp.float32),
        pltpu.VMEM((_N,), jnp.float32),
        pltpu.VMEM((_N,), jnp.float32),
        pltpu.VMEM((_SEC + 16,), jnp.int32),
        pltpu.VMEM((_SEC + 16,), jnp.int32),
        pltpu.VMEM((_CH, _K), jnp.int32),
        pltpu.VMEM((_CH, _K), jnp.int32),
        pltpu.VMEM((128, _CH2), jnp.float32),
        pltpu.VMEM((128, _CH2), jnp.float32),
        pltpu.VMEM((128,), jnp.float32),
        pltpu.VMEM((128,), jnp.float32),
        pltpu.VMEM((128,), jnp.float32),
        pltpu.VMEM((128,), jnp.float32),
        pltpu.VMEM((144,), jnp.float32),
        pltpu.VMEM((1024,), jnp.float32),
        pltpu.VMEM_SHARED((_N,), jnp.float32),
        pltpu.VMEM_SHARED((_NPAD, _CH2), jnp.float32),
        pltpu.SemaphoreType.DMA,
        pltpu.SemaphoreType.DMA,
        pltpu.SemaphoreType.DMA,
        pltpu.SemaphoreType.DMA,
        pltpu.SemaphoreType.DMA,
        pltpu.SemaphoreType.DMA,
        pltpu.SemaphoreType.DMA,
        pltpu.SemaphoreType.DMA,
    ],
)(_sc_body)


_BLK = 1000


def _emit_h_av(h, as_ref, ad_ref, h_ref, av_ref):
    h_ref[0] = h[:, :_CH2]
    h_ref[1] = h[:, _CH2:]
    av_ref[0, :, 0] = jnp.sum(h * as_ref[...], axis=1)
    av_ref[1, :, 0] = jnp.sum(h * ad_ref[...], axis=1)


def _tc_first_body(x_ref, w0_ref, b0_ref, w1_ref, as_ref, ad_ref,
                   out_ref, h_ref, av_ref):
    t = jnp.dot(x_ref[...], w0_ref[...],
                preferred_element_type=jnp.float32) + b0_ref[...]
    o = jax.nn.softplus(t)
    out_ref[...] = o
    h = jnp.dot(o, w1_ref[...], preferred_element_type=jnp.float32)
    _emit_h_av(h, as_ref, ad_ref, h_ref, av_ref)


def _tc_mid_body(o_ref, a_ref, b_ref, w_ref, as_ref, ad_ref,
                 out_ref, h_ref, av_ref):
    o = (o_ref[...] + jnp.concatenate([a_ref[0], a_ref[1]], axis=-1)
         + b_ref[...])
    out_ref[...] = o
    h = jnp.dot(o, w_ref[...], preferred_element_type=jnp.float32)
    _emit_h_av(h, as_ref, ad_ref, h_ref, av_ref)


def _tc_final_body(o_ref, a_ref, b_ref, out_ref):
    out_ref[...] = (o_ref[...]
                    + jnp.concatenate([a_ref[0], a_ref[1]], axis=-1)
                    + b_ref[...])


def _row_spec():
    return pl.BlockSpec((_BLK, _C), lambda i: (i, 0))


def _aggr_spec():
    return pl.BlockSpec((2, _BLK, _CH2), lambda i: (0, i, 0))


def _h_spec():
    return pl.BlockSpec((2, _BLK, _CH2), lambda i: (0, i, 0))


def _av_spec():
    return pl.BlockSpec((2, _BLK, 1), lambda i: (0, i, 0))


def _w_spec():
    return pl.BlockSpec((_C, _C), lambda i: (0, 0))


def _b_spec():
    return pl.BlockSpec((1, _C), lambda i: (0, 0))


_out_h_av_shapes = [jax.ShapeDtypeStruct((_N, _C), jnp.float32),
                    jax.ShapeDtypeStruct((2, _N, _CH2), jnp.float32),
                    jax.ShapeDtypeStruct((2, _N, 1), jnp.float32)]

_tc_first = pl.pallas_call(
    _tc_first_body,
    grid=(_N // _BLK,),
    in_specs=[_row_spec(), _w_spec(), _b_spec(), _w_spec(),
              _b_spec(), _b_spec()],
    out_specs=[_row_spec(), _h_spec(), _av_spec()],
    out_shape=_out_h_av_shapes,
)

_tc_mid = pl.pallas_call(
    _tc_mid_body,
    grid=(_N // _BLK,),
    in_specs=[_row_spec(), _aggr_spec(), _b_spec(), _w_spec(),
              _b_spec(), _b_spec()],
    out_specs=[_row_spec(), _h_spec(), _av_spec()],
    out_shape=_out_h_av_shapes,
)

_tc_final = pl.pallas_call(
    _tc_final_body,
    grid=(_N // _BLK,),
    in_specs=[_row_spec(), _aggr_spec(), _b_spec()],
    out_specs=_row_spec(),
    out_shape=jax.ShapeDtypeStruct((_N, _C), jnp.float32),
)


def kernel(x, edge_index, edge_weight, edge_attr, lin0_W, lin0_b,
           conv_W, att_src, att_dst, conv_bias):
    del edge_weight, edge_attr
    ei = edge_index.astype(jnp.int32)
    ei2 = ei.reshape(2, _E // _K, _K)

    out0, h1, av1 = _tc_first(x, lin0_W, lin0_b.reshape(1, _C), conv_W[0],
                              att_src[0].reshape(1, _C),
                              att_dst[0].reshape(1, _C))
    aggr1 = _sc_edges(ei, ei2, av1.reshape(2, _N), h1)
    out1, h2, av2 = _tc_mid(out0, aggr1, conv_bias[0].reshape(1, _C),
                            conv_W[1],
                            att_src[1].reshape(1, _C),
                            att_dst[1].reshape(1, _C))
    aggr2 = _sc_edges(ei, ei2, av2.reshape(2, _N), h2)
    return _tc_final(out1, aggr2, conv_bias[1].reshape(1, _C))
